# Initial kernel scaffold; baseline (speedup 1.0000x reference)
import jax
import jax.numpy as jnp
from jax.experimental import pallas as pl


def kernel(h, edges, edge_attr, params):
    raise NotImplementedError("write your pallas kernel here")


# speedup vs baseline: 1.8632x; 1.8632x over previous
import functools

import jax
import jax.numpy as jnp
from jax import lax
from jax.experimental import pallas as pl
from jax.experimental.pallas import tpu as pltpu
from jax.experimental.pallas import tpu_sc as plsc

HID = 48
N_LAYERS = 4
N_NODES = 5
N_FRAME = 8

_SIGNS = jnp.array(
    [[1, 1, 1], [1, 1, -1], [1, -1, 1], [1, -1, -1],
     [-1, 1, 1], [-1, 1, -1], [-1, -1, 1], [-1, -1, -1]],
    dtype=jnp.float32)


def _create_frame(nodes, n_nodes):
    pnts = nodes[:, :3].reshape(-1, n_nodes, 3).transpose(0, 2, 1)
    v = nodes[:, 3:].reshape(-1, n_nodes, 3).transpose(0, 2, 1)
    center = pnts.mean(axis=2, keepdims=True)
    pc = pnts - center
    R = jnp.matmul(pc, pc.transpose(0, 2, 1))
    _, V = jnp.linalg.eigh(lax.stop_gradient(R))
    F_ops = _SIGNS[None, :, None, :] * V[:, None, :, :]
    framed_input = jnp.einsum('boij,bpj->bopi', F_ops.transpose(0, 1, 3, 2),
                              pc.transpose(0, 2, 1))
    framed_v = jnp.einsum('boij,bpj->bopi', F_ops.transpose(0, 1, 3, 2),
                          v.transpose(0, 2, 1))
    framed_input = framed_input.transpose(1, 0, 2, 3).reshape(-1, 3)
    framed_v = framed_v.transpose(1, 0, 2, 3).reshape(-1, 3)
    out = jnp.concatenate([framed_input, framed_v], axis=1)
    return out, lax.stop_gradient(F_ops), lax.stop_gradient(center)


def _create_latent_frame(pnts, n_nodes):
    pnts = pnts.transpose(0, 2, 1)
    center = pnts.mean(axis=2, keepdims=True)
    pc = pnts - center
    R = jnp.matmul(pc, pc.transpose(0, 2, 1))
    _, V = jnp.linalg.eigh(lax.stop_gradient(R))
    F_ops = _SIGNS[None, :, None, :] * V[:, None, :, :]
    fi = jnp.einsum('boij,bpj->bopi', F_ops.transpose(0, 1, 3, 2),
                    pc.transpose(0, 2, 1))
    fi = fi.transpose(1, 0, 2, 3)
    fi = fi.reshape(fi.shape[0], fi.shape[1], n_nodes, -1, 3)
    fi = fi.reshape(-1, fi.shape[-2] * 3)
    return fi, lax.stop_gradient(F_ops)


def _invert_frame(pnts, F_ops, n_nodes, center):
    pnts = pnts.reshape(8, -1, n_nodes, 3).transpose(1, 0, 2, 3)
    fi = jnp.einsum('boij,bopj->bopi', F_ops, pnts).mean(axis=1)
    fi = fi + center.transpose(0, 2, 1)
    return fi.reshape(-1, 3)


def _invert_latent_frame(pnts, F_ops, batch_size, n_nodes):
    pnts = pnts.reshape(8, batch_size, n_nodes, -1, 3).transpose(1, 0, 2, 3, 4)
    fi = jnp.einsum('boij,bopfj->bopfi', F_ops, pnts).mean(axis=1)
    return fi.reshape(batch_size, -1, 3)


def _pack(h_exp, n):
    return h_exp.reshape(N_FRAME, n, HID).transpose(1, 0, 2).reshape(
        n, N_FRAME, 1, HID)


def _unpack(h4, n):
    return h4.reshape(n, N_FRAME, HID).transpose(1, 0, 2).reshape(-1, HID)


_SUB = 128
_PKD = N_FRAME * HID


def _sc_gather(table, idx, n_iter):
    mesh = plsc.VectorSubcoreMesh(core_axis_name="c", subcore_axis_name="s")
    nc = mesh.num_cores
    nw = nc * mesh.num_subcores
    m = idx.shape[0]
    d = table.shape[1]
    per_w = m // nw

    @functools.partial(
        pl.kernel,
        mesh=mesh,
        out_type=jax.ShapeDtypeStruct((m, d), jnp.float32),
        scratch_types=[
            pltpu.VMEM((_SUB,), jnp.int32),
            pltpu.VMEM((_SUB, d), jnp.float32),
            pltpu.SemaphoreType.DMA,
        ],
    )
    def k(table_hbm, idx_hbm, out_hbm, idx_v, rows_v, sem):
        wid = lax.axis_index("s") * nc + lax.axis_index("c")
        base = wid * per_w

        @pl.loop(0, n_iter)
        def _(i):
            off = base + i * _SUB
            pltpu.sync_copy(idx_hbm.at[pl.ds(off, _SUB)], idx_v)
            pltpu.async_copy(table_hbm.at[idx_v], rows_v, sem).wait()
            pltpu.sync_copy(rows_v, out_hbm.at[pl.ds(off, _SUB)])

    return k(table, idx)


_BE = 128
_WIN = 256


def _silu(x):
    return x * jax.nn.sigmoid(x)


def _edge_kernel_body(sref, gr_ref, gc_ref, ea_ref, cid_ref,
                      a_ref, b_ref, c_ref, be1_ref, w2_ref, be2_ref, out_ref,
                      *, bpf):
    b = pl.program_id(0)

    @pl.when(b % bpf == 0)
    def _():
        out_ref[...] = jnp.zeros(out_ref.shape, out_ref.dtype)

    gr = gr_ref[...].reshape(_BE, HID)
    gc = gc_ref[...].reshape(_BE, HID)
    ea = ea_ref[...]
    pre = (jnp.dot(gr, a_ref[...], preferred_element_type=jnp.float32)
           + jnp.dot(gc, b_ref[...], preferred_element_type=jnp.float32)
           + ea[:, 0:1] * c_ref[0:1, :] + ea[:, 1:2] * c_ref[1:2, :]
           + be1_ref[...])
    e1 = _silu(pre)
    e2 = _silu(jnp.dot(e1, w2_ref[...], preferred_element_type=jnp.float32)
               + be2_ref[...])
    base = sref[b % bpf]
    loc = cid_ref[...].reshape(1, _BE) - base
    oh = (loc == lax.broadcasted_iota(jnp.int32, (_WIN, _BE), 0)
          ).astype(jnp.float32)
    partial = jnp.dot(oh, e2, preferred_element_type=jnp.float32)
    out_ref[pl.ds(base, _WIN), 0, 0, :] = (
        out_ref[pl.ds(base, _WIN), 0, 0, :] + partial)


def _edge_layer(g4, ea_s, cid3, cbase_al, p, e_base, nseg_pad):
    bpf = e_base // _BE
    nb = N_FRAME * bpf
    a_w = p['We1'][:HID]
    b_w = p['We1'][HID:2 * HID]
    c_w = p['We1'][2 * HID:]
    grid_spec = pltpu.PrefetchScalarGridSpec(
        num_scalar_prefetch=1,
        grid=(nb,),
        in_specs=[
            pl.BlockSpec((_BE, 1, 1, HID), lambda b, s: (b % bpf, b // bpf, 0, 0)),
            pl.BlockSpec((_BE, 1, 1, HID),
                         lambda b, s: (bpf + b % bpf, b // bpf, 0, 0)),
            pl.BlockSpec((_BE, 2), lambda b, s: (b % bpf, 0)),
            pl.BlockSpec((1, 1, _BE), lambda b, s: (b % bpf, 0, 0)),
            pl.BlockSpec((HID, HID), lambda b, s: (0, 0)),
            pl.BlockSpec((HID, HID), lambda b, s: (0, 0)),
            pl.BlockSpec((2, HID), lambda b, s: (0, 0)),
            pl.BlockSpec((1, HID), lambda b, s: (0, 0)),
            pl.BlockSpec((HID, HID), lambda b, s: (0, 0)),
            pl.BlockSpec((1, HID), lambda b, s: (0, 0)),
        ],
        out_specs=pl.BlockSpec((nseg_pad, 1, 1, HID),
                               lambda b, s: (0, b // bpf, 0, 0)),
    )
    return pl.pallas_call(
        functools.partial(_edge_kernel_body, bpf=bpf),
        grid_spec=grid_spec,
        out_shape=jax.ShapeDtypeStruct((nseg_pad, N_FRAME, 1, HID),
                                       jnp.float32),
        compiler_params=pltpu.CompilerParams(
            dimension_semantics=("arbitrary",)),
    )(cbase_al, g4, g4, ea_s, cid3,
      a_w, b_w, c_w, p['be1'].reshape(1, HID),
      p['We2'], p['be2'].reshape(1, HID))


def _node_kernel_body(h_ref, agg_ref, wa_ref, wb_ref, b1_ref, w2_ref, b2_ref,
                      out_ref, *, bn):
    h = h_ref[...].reshape(bn, HID)
    agg = agg_ref[...].reshape(bn, HID)
    t = _silu(
        jnp.dot(h, wa_ref[...], preferred_element_type=jnp.float32)
        + jnp.dot(agg, wb_ref[...], preferred_element_type=jnp.float32)
        + b1_ref[...])
    out = (jnp.dot(t, w2_ref[...], preferred_element_type=jnp.float32)
           + b2_ref[...])
    out_ref[...] = out.reshape(bn, 1, 1, HID)


def _node_layer(h4, agg4, p, n, bn):
    wa = p['Wn1'][:HID]
    wb = p['Wn1'][HID:]
    return pl.pallas_call(
        functools.partial(_node_kernel_body, bn=bn),
        grid=(N_FRAME, n // bn),
        in_specs=[
            pl.BlockSpec((bn, 1, 1, HID), lambda k, i: (i, k, 0, 0)),
            pl.BlockSpec((bn, 1, 1, HID), lambda k, i: (i, k, 0, 0)),
            pl.BlockSpec((HID, HID), lambda k, i: (0, 0)),
            pl.BlockSpec((HID, HID), lambda k, i: (0, 0)),
            pl.BlockSpec((1, HID), lambda k, i: (0, 0)),
            pl.BlockSpec((HID, HID), lambda k, i: (0, 0)),
            pl.BlockSpec((1, HID), lambda k, i: (0, 0)),
        ],
        out_specs=pl.BlockSpec((bn, 1, 1, HID), lambda k, i: (i, k, 0, 0)),
        out_shape=jax.ShapeDtypeStruct((n, N_FRAME, 1, HID), jnp.float32),
    )(h4, agg4, wa, wb, p['bn1'].reshape(1, HID), p['Wn2'],
      p['bn2'].reshape(1, HID))


def _embed_kernel_body(x_ref, w_ref, b_ref, out_ref, *, bn):
    x = x_ref[...].reshape(bn, 6)
    out = (jnp.dot(x, w_ref[...], preferred_element_type=jnp.float32)
           + b_ref[...])
    out_ref[...] = out.reshape(bn, 1, 1, HID)


def _embed(x4, w, b, n, bn):
    return pl.pallas_call(
        functools.partial(_embed_kernel_body, bn=bn),
        grid=(N_FRAME, n // bn),
        in_specs=[
            pl.BlockSpec((bn, 1, 1, 6), lambda k, i: (i, k, 0, 0)),
            pl.BlockSpec((6, HID), lambda k, i: (0, 0)),
            pl.BlockSpec((1, HID), lambda k, i: (0, 0)),
        ],
        out_specs=pl.BlockSpec((bn, 1, 1, HID), lambda k, i: (i, k, 0, 0)),
        out_shape=jax.ShapeDtypeStruct((n, N_FRAME, 1, HID), jnp.float32),
    )(x4, w, b.reshape(1, HID))


def _mlp2_kernel_body(x_ref, w1_ref, b1_ref, w2_ref, b2_ref, out_ref):
    t = _silu(
        jnp.dot(x_ref[...], w1_ref[...], preferred_element_type=jnp.float32)
        + b1_ref[...])
    out_ref[...] = (jnp.dot(t, w2_ref[...], preferred_element_type=jnp.float32)
                    + b2_ref[...])


def _mlp2(x, w1, b1, w2, b2, bn):
    nt, k = x.shape
    h1 = w1.shape[1]
    h2 = w2.shape[1]
    return pl.pallas_call(
        _mlp2_kernel_body,
        grid=(nt // bn,),
        in_specs=[
            pl.BlockSpec((bn, k), lambda b: (b, 0)),
            pl.BlockSpec((k, h1), lambda b: (0, 0)),
            pl.BlockSpec((1, h1), lambda b: (0, 0)),
            pl.BlockSpec((h1, h2), lambda b: (0, 0)),
            pl.BlockSpec((1, h2), lambda b: (0, 0)),
        ],
        out_specs=pl.BlockSpec((bn, h2), lambda b: (b, 0)),
        out_shape=jax.ShapeDtypeStruct((nt, h2), jnp.float32),
    )(x, w1, b1.reshape(1, h1), w2, b2.reshape(1, h2))


def kernel(h, edges, edge_attr, params):
    n = h.shape[0]
    batch_size = n // N_NODES
    e_base = edges.shape[1]
    nt = N_FRAME * n
    bn = 1000 if n % 1000 == 0 else n
    nseg_pad = -(-(n + 2 * _WIN) // 8) * 8

    nw = 32
    m_edge = 2 * e_base
    n_it_e = -(-m_edge // (nw * _SUB))
    m_edge_pad = nw * _SUB * n_it_e
    n_it_n = -(-n // (nw * _SUB))
    m_node_pad = nw * _SUB * n_it_n

    row0 = edges[0].astype(jnp.int32)
    col0 = edges[1].astype(jnp.int32)

    perm0 = jnp.argsort(row0)
    rs0 = row0[perm0]
    cs0 = col0[perm0]
    ea_s = edge_attr[perm0]
    newseg = jnp.concatenate(
        [jnp.ones((1,), jnp.int32), (rs0[1:] != rs0[:-1]).astype(jnp.int32)])
    cid0 = jnp.cumsum(newseg) - 1
    cid3 = cid0.reshape(e_base // _BE, 1, _BE)
    cbase_al = ((cid0[::_BE] // _BE) * _BE).astype(jnp.int32)

    idx_edge = jnp.concatenate(
        [rs0, cs0, jnp.zeros((m_edge_pad - m_edge,), jnp.int32)])

    has0 = jnp.zeros((n,), jnp.int32).at[rs0].set(1)
    pos0 = jnp.cumsum(has0) - has0
    idx_node = jnp.where(has0 == 1, pos0, nseg_pad - 1).astype(jnp.int32)
    idx_node = jnp.concatenate(
        [idx_node, jnp.zeros((m_node_pad - n,), jnp.int32)])

    framed, f_ops, center = _create_frame(h, N_NODES)
    framed4 = framed.reshape(N_FRAME, n, 6).transpose(1, 0, 2).reshape(
        n, N_FRAME, 1, 6)
    hcur4 = _embed(framed4, params['W_emb'], params['b_emb'], n, bn)

    for i in range(N_LAYERS):
        p = params['gcl_%d' % i]
        g = _sc_gather(hcur4.reshape(n, _PKD), idx_edge, n_it_e)
        g4 = g.reshape(m_edge_pad, N_FRAME, 1, HID)
        agg_c = _edge_layer(g4, ea_s, cid3, cbase_al, p, e_base, nseg_pad)
        agg = _sc_gather(agg_c.reshape(nseg_pad, _PKD), idx_node, n_it_n)
        agg4 = agg.reshape(m_node_pad, N_FRAME, 1, HID)[:n]
        hcur4 = _node_layer(hcur4, agg4, p, n, bn)
        if i < N_LAYERS - 1:
            lat = _invert_latent_frame(_unpack(hcur4, n), f_ops, batch_size,
                                       N_NODES)
            hlat, f_ops = _create_latent_frame(lat, N_NODES)
            hcur4 = _pack(hlat, n)

    hdec = _unpack(hcur4, n)
    dec = _mlp2(hdec, params['Wd1'], params['bd1'],
                jnp.pad(params['Wd2'], ((0, 0), (0, 5))),
                jnp.pad(params['bd2'], (0, 5)), bn)
    out = _invert_frame(dec[:, :3], f_ops, N_NODES, center)
    return out


# speedup vs baseline: 2.1421x; 1.1497x over previous
import functools

import jax
import jax.numpy as jnp
from jax import lax
from jax.experimental import pallas as pl
from jax.experimental.pallas import tpu as pltpu
from jax.experimental.pallas import tpu_sc as plsc

HID = 48
N_LAYERS = 4
N_NODES = 5
N_FRAME = 8

_SIGNS = jnp.array(
    [[1, 1, 1], [1, 1, -1], [1, -1, 1], [1, -1, -1],
     [-1, 1, 1], [-1, 1, -1], [-1, -1, 1], [-1, -1, -1]],
    dtype=jnp.float32)


def _create_frame(nodes, n_nodes):
    pnts = nodes[:, :3].reshape(-1, n_nodes, 3).transpose(0, 2, 1)
    v = nodes[:, 3:].reshape(-1, n_nodes, 3).transpose(0, 2, 1)
    center = pnts.mean(axis=2, keepdims=True)
    pc = pnts - center
    R = jnp.matmul(pc, pc.transpose(0, 2, 1))
    _, V = jnp.linalg.eigh(lax.stop_gradient(R))
    F_ops = _SIGNS[None, :, None, :] * V[:, None, :, :]
    framed_input = jnp.einsum('boij,bpj->bopi', F_ops.transpose(0, 1, 3, 2),
                              pc.transpose(0, 2, 1))
    framed_v = jnp.einsum('boij,bpj->bopi', F_ops.transpose(0, 1, 3, 2),
                          v.transpose(0, 2, 1))
    framed_input = framed_input.transpose(1, 0, 2, 3).reshape(-1, 3)
    framed_v = framed_v.transpose(1, 0, 2, 3).reshape(-1, 3)
    out = jnp.concatenate([framed_input, framed_v], axis=1)
    return out, lax.stop_gradient(F_ops), lax.stop_gradient(center)


def _create_latent_frame(pnts, n_nodes):
    pnts = pnts.transpose(0, 2, 1)
    center = pnts.mean(axis=2, keepdims=True)
    pc = pnts - center
    R = jnp.matmul(pc, pc.transpose(0, 2, 1))
    _, V = jnp.linalg.eigh(lax.stop_gradient(R))
    F_ops = _SIGNS[None, :, None, :] * V[:, None, :, :]
    fi = jnp.einsum('boij,bpj->bopi', F_ops.transpose(0, 1, 3, 2),
                    pc.transpose(0, 2, 1))
    fi = fi.transpose(1, 0, 2, 3)
    fi = fi.reshape(fi.shape[0], fi.shape[1], n_nodes, -1, 3)
    fi = fi.reshape(-1, fi.shape[-2] * 3)
    return fi, lax.stop_gradient(F_ops)


def _invert_frame(pnts, F_ops, n_nodes, center):
    pnts = pnts.reshape(8, -1, n_nodes, 3).transpose(1, 0, 2, 3)
    fi = jnp.einsum('boij,bopj->bopi', F_ops, pnts).mean(axis=1)
    fi = fi + center.transpose(0, 2, 1)
    return fi.reshape(-1, 3)


def _invert_latent_frame(pnts, F_ops, batch_size, n_nodes):
    pnts = pnts.reshape(8, batch_size, n_nodes, -1, 3).transpose(1, 0, 2, 3, 4)
    fi = jnp.einsum('boij,bopfj->bopfi', F_ops, pnts).mean(axis=1)
    return fi.reshape(batch_size, -1, 3)


def _pack(h_exp, n):
    return h_exp.reshape(N_FRAME, n, HID).transpose(1, 0, 2).reshape(
        n, N_FRAME, 1, HID)


def _unpack(h4, n):
    return h4.reshape(n, N_FRAME, HID).transpose(1, 0, 2).reshape(-1, HID)


_SUB = 128
_PKD = N_FRAME * HID


def _sc_gather(table, idx, n_iter):
    mesh = plsc.VectorSubcoreMesh(core_axis_name="c", subcore_axis_name="s")
    nc = mesh.num_cores
    nw = nc * mesh.num_subcores
    m = idx.shape[0]
    d = table.shape[1]
    per_w = m // nw

    @functools.partial(
        pl.kernel,
        mesh=mesh,
        out_type=jax.ShapeDtypeStruct((m, d), jnp.float32),
        scratch_types=[
            pltpu.VMEM((_SUB,), jnp.int32),
            pltpu.VMEM((_SUB, d), jnp.float32),
            pltpu.SemaphoreType.DMA,
        ],
    )
    def k(table_hbm, idx_hbm, out_hbm, idx_v, rows_v, sem):
        wid = lax.axis_index("s") * nc + lax.axis_index("c")
        base = wid * per_w

        @pl.loop(0, n_iter)
        def _(i):
            off = base + i * _SUB
            pltpu.sync_copy(idx_hbm.at[pl.ds(off, _SUB)], idx_v)
            pltpu.async_copy(table_hbm.at[idx_v], rows_v, sem).wait()
            pltpu.sync_copy(rows_v, out_hbm.at[pl.ds(off, _SUB)])

    return k(table, idx)


_BE = 128
_WIN = 256


def _silu(x):
    return x * jax.nn.sigmoid(x)


def _edge_kernel_body(sref, gr_ref, gc_ref, ea_ref, cid_ref,
                      a_ref, b_ref, c_ref, be1_ref, w2_ref, be2_ref, out_ref):
    b = pl.program_id(0)

    @pl.when(b == 0)
    def _():
        out_ref[...] = jnp.zeros(out_ref.shape, out_ref.dtype)

    ea = ea_ref[...]
    eac = (ea[:, 0:1] * c_ref[0:1, :] + ea[:, 1:2] * c_ref[1:2, :]
           + be1_ref[...])
    base = sref[b] * 8
    loc = cid_ref[...].reshape(1, _BE) - base
    oh = (loc == lax.broadcasted_iota(jnp.int32, (_WIN, _BE), 0)
          ).astype(jnp.float32)
    e2s = []
    for k in range(N_FRAME):
        gr = gr_ref[:, k, 0, :]
        gc = gc_ref[:, k, 0, :]
        pre = (jnp.dot(gr, a_ref[...], preferred_element_type=jnp.float32)
               + jnp.dot(gc, b_ref[...], preferred_element_type=jnp.float32)
               + eac)
        e1 = _silu(pre)
        e2s.append(
            _silu(jnp.dot(e1, w2_ref[...], preferred_element_type=jnp.float32)
                  + be2_ref[...]))
    e2all = jnp.concatenate(e2s, axis=1)
    partial = jnp.dot(oh, e2all, preferred_element_type=jnp.float32)
    out_ref[pl.ds(base, _WIN), :] = out_ref[pl.ds(base, _WIN), :] + partial


def _edge_layer(g4, ea_s, cid3, cbase_al, p, e_base, nseg_pad):
    bpf = e_base // _BE
    a_w = p['We1'][:HID]
    b_w = p['We1'][HID:2 * HID]
    c_w = p['We1'][2 * HID:]
    grid_spec = pltpu.PrefetchScalarGridSpec(
        num_scalar_prefetch=1,
        grid=(bpf,),
        in_specs=[
            pl.BlockSpec((_BE, N_FRAME, 1, HID), lambda b, s: (b, 0, 0, 0)),
            pl.BlockSpec((_BE, N_FRAME, 1, HID),
                         lambda b, s: (bpf + b, 0, 0, 0)),
            pl.BlockSpec((_BE, 2), lambda b, s: (b, 0)),
            pl.BlockSpec((1, 1, _BE), lambda b, s: (b, 0, 0)),
            pl.BlockSpec((HID, HID), lambda b, s: (0, 0)),
            pl.BlockSpec((HID, HID), lambda b, s: (0, 0)),
            pl.BlockSpec((2, HID), lambda b, s: (0, 0)),
            pl.BlockSpec((1, HID), lambda b, s: (0, 0)),
            pl.BlockSpec((HID, HID), lambda b, s: (0, 0)),
            pl.BlockSpec((1, HID), lambda b, s: (0, 0)),
        ],
        out_specs=pl.BlockSpec((nseg_pad, _PKD), lambda b, s: (0, 0)),
    )
    return pl.pallas_call(
        _edge_kernel_body,
        grid_spec=grid_spec,
        out_shape=jax.ShapeDtypeStruct((nseg_pad, _PKD), jnp.float32),
        compiler_params=pltpu.CompilerParams(
            dimension_semantics=("arbitrary",)),
    )(cbase_al, g4, g4, ea_s, cid3,
      a_w, b_w, c_w, p['be1'].reshape(1, HID),
      p['We2'], p['be2'].reshape(1, HID))


def _node_kernel_body(h_ref, agg_ref, wa_ref, wb_ref, b1_ref, w2_ref, b2_ref,
                      out_ref, *, bn):
    h = h_ref[...].reshape(bn, HID)
    agg = agg_ref[...].reshape(bn, HID)
    t = _silu(
        jnp.dot(h, wa_ref[...], preferred_element_type=jnp.float32)
        + jnp.dot(agg, wb_ref[...], preferred_element_type=jnp.float32)
        + b1_ref[...])
    out = (jnp.dot(t, w2_ref[...], preferred_element_type=jnp.float32)
           + b2_ref[...])
    out_ref[...] = out.reshape(bn, 1, 1, HID)


def _node_layer(h4, agg4, p, n, bn):
    wa = p['Wn1'][:HID]
    wb = p['Wn1'][HID:]
    return pl.pallas_call(
        functools.partial(_node_kernel_body, bn=bn),
        grid=(N_FRAME, n // bn),
        in_specs=[
            pl.BlockSpec((bn, 1, 1, HID), lambda k, i: (i, k, 0, 0)),
            pl.BlockSpec((bn, 1, 1, HID), lambda k, i: (i, k, 0, 0)),
            pl.BlockSpec((HID, HID), lambda k, i: (0, 0)),
            pl.BlockSpec((HID, HID), lambda k, i: (0, 0)),
            pl.BlockSpec((1, HID), lambda k, i: (0, 0)),
            pl.BlockSpec((HID, HID), lambda k, i: (0, 0)),
            pl.BlockSpec((1, HID), lambda k, i: (0, 0)),
        ],
        out_specs=pl.BlockSpec((bn, 1, 1, HID), lambda k, i: (i, k, 0, 0)),
        out_shape=jax.ShapeDtypeStruct((n, N_FRAME, 1, HID), jnp.float32),
    )(h4, agg4, wa, wb, p['bn1'].reshape(1, HID), p['Wn2'],
      p['bn2'].reshape(1, HID))


def _embed_kernel_body(x_ref, w_ref, b_ref, out_ref, *, bn):
    x = x_ref[...].reshape(bn, 6)
    out = (jnp.dot(x, w_ref[...], preferred_element_type=jnp.float32)
           + b_ref[...])
    out_ref[...] = out.reshape(bn, 1, 1, HID)


def _embed(x4, w, b, n, bn):
    return pl.pallas_call(
        functools.partial(_embed_kernel_body, bn=bn),
        grid=(N_FRAME, n // bn),
        in_specs=[
            pl.BlockSpec((bn, 1, 1, 6), lambda k, i: (i, k, 0, 0)),
            pl.BlockSpec((6, HID), lambda k, i: (0, 0)),
            pl.BlockSpec((1, HID), lambda k, i: (0, 0)),
        ],
        out_specs=pl.BlockSpec((bn, 1, 1, HID), lambda k, i: (i, k, 0, 0)),
        out_shape=jax.ShapeDtypeStruct((n, N_FRAME, 1, HID), jnp.float32),
    )(x4, w, b.reshape(1, HID))


def _mlp2_kernel_body(x_ref, w1_ref, b1_ref, w2_ref, b2_ref, out_ref):
    t = _silu(
        jnp.dot(x_ref[...], w1_ref[...], preferred_element_type=jnp.float32)
        + b1_ref[...])
    out_ref[...] = (jnp.dot(t, w2_ref[...], preferred_element_type=jnp.float32)
                    + b2_ref[...])


def _mlp2(x, w1, b1, w2, b2, bn):
    nt, k = x.shape
    h1 = w1.shape[1]
    h2 = w2.shape[1]
    return pl.pallas_call(
        _mlp2_kernel_body,
        grid=(nt // bn,),
        in_specs=[
            pl.BlockSpec((bn, k), lambda b: (b, 0)),
            pl.BlockSpec((k, h1), lambda b: (0, 0)),
            pl.BlockSpec((1, h1), lambda b: (0, 0)),
            pl.BlockSpec((h1, h2), lambda b: (0, 0)),
            pl.BlockSpec((1, h2), lambda b: (0, 0)),
        ],
        out_specs=pl.BlockSpec((bn, h2), lambda b: (b, 0)),
        out_shape=jax.ShapeDtypeStruct((nt, h2), jnp.float32),
    )(x, w1, b1.reshape(1, h1), w2, b2.reshape(1, h2))


def kernel(h, edges, edge_attr, params):
    n = h.shape[0]
    batch_size = n // N_NODES
    e_base = edges.shape[1]
    nt = N_FRAME * n
    bn = 1000 if n % 1000 == 0 else n
    nseg_pad = -(-(n + 2 * _WIN) // 8) * 8

    nw = 32
    m_edge = 2 * e_base
    n_it_e = -(-m_edge // (nw * _SUB))
    m_edge_pad = nw * _SUB * n_it_e
    n_it_n = -(-n // (nw * _SUB))
    m_node_pad = nw * _SUB * n_it_n

    row0 = edges[0].astype(jnp.int32)
    col0 = edges[1].astype(jnp.int32)

    perm0 = jnp.argsort(row0)
    rs0 = row0[perm0]
    cs0 = col0[perm0]
    ea_s = edge_attr[perm0]
    newseg = jnp.concatenate(
        [jnp.ones((1,), jnp.int32), (rs0[1:] != rs0[:-1]).astype(jnp.int32)])
    cid0 = jnp.cumsum(newseg) - 1
    cid3 = cid0.reshape(e_base // _BE, 1, _BE)
    cbase_al = ((cid0[::_BE] // _BE) * (_BE // 8)).astype(jnp.int32)

    idx_edge = jnp.concatenate(
        [rs0, cs0, jnp.zeros((m_edge_pad - m_edge,), jnp.int32)])

    has0 = jnp.zeros((n,), jnp.int32).at[rs0].set(1)
    pos0 = jnp.cumsum(has0) - has0
    idx_node = jnp.where(has0 == 1, pos0, nseg_pad - 1).astype(jnp.int32)
    idx_node = jnp.concatenate(
        [idx_node, jnp.zeros((m_node_pad - n,), jnp.int32)])

    framed, f_ops, center = _create_frame(h, N_NODES)
    framed4 = framed.reshape(N_FRAME, n, 6).transpose(1, 0, 2).reshape(
        n, N_FRAME, 1, 6)
    hcur4 = _embed(framed4, params['W_emb'], params['b_emb'], n, bn)

    for i in range(N_LAYERS):
        p = params['gcl_%d' % i]
        g = _sc_gather(hcur4.reshape(n, _PKD), idx_edge, n_it_e)
        g4 = g.reshape(m_edge_pad, N_FRAME, 1, HID)
        agg_c = _edge_layer(g4, ea_s, cid3, cbase_al, p, e_base, nseg_pad)
        agg = _sc_gather(agg_c, idx_node, n_it_n)
        agg4 = agg.reshape(m_node_pad, N_FRAME, 1, HID)[:n]
        hcur4 = _node_layer(hcur4, agg4, p, n, bn)
        if i < N_LAYERS - 1:
            lat = _invert_latent_frame(_unpack(hcur4, n), f_ops, batch_size,
                                       N_NODES)
            hlat, f_ops = _create_latent_frame(lat, N_NODES)
            hcur4 = _pack(hlat, n)

    hdec = _unpack(hcur4, n)
    dec = _mlp2(hdec, params['Wd1'], params['bd1'],
                jnp.pad(params['Wd2'], ((0, 0), (0, 5))),
                jnp.pad(params['bd2'], (0, 5)), bn)
    out = _invert_frame(dec[:, :3], f_ops, N_NODES, center)
    return out
